# baseline (speedup 1.0000x reference)
import functools

import jax
import jax.numpy as jnp
from jax import lax
from jax.experimental import pallas as pl
from jax.experimental.pallas import tpu as pltpu
from jax.experimental.pallas import tpu_sc as plsc

_K = 6
_KS = 245
_NUM_CTX = sum(5 ** i for i in range(_K + 1))
_B, _L = 4096, 50
_NCOL = _L + 1
_NCOLP = _NCOL + 1
_KSP = 256

_NC, _NS = 2, 16
_NW = _NC * _NS
_BW = _B // _NW
_BC = 2
_NCHUNK = _BW // _BC
_CHI = _BC * _NCOLP
_IDXW = _BW * _NCOLP


def _inds_body(x_ref, out_ref):
    x = x_ref[:]
    offs = [(5 ** m - 1) // 4 for m in range(_K + 1)]
    cols = []
    v = jnp.zeros((_B, 1), jnp.int32)
    cols.append(v + offs[0])
    for i in range(1, _K):
        v = v * 5 + x[:, i - 1:i]
        cols.append(v + offs[i])
    wide = _L - _K + 1
    big = jnp.zeros((_B, wide), jnp.int32)
    for j in range(_K):
        big = big * 5 + x[:, j:j + wide]
    cols.append(big + offs[_K])
    cols.append(jnp.zeros((_B, 1), jnp.int32))
    out_ref[:] = jnp.concatenate(cols, axis=1)


def _compute_inds(x):
    return pl.pallas_call(
        _inds_body,
        out_shape=jax.ShapeDtypeStruct((_B, _NCOLP), jnp.int32),
    )(x)


def _bias_body(e_ref, b_ref, o_ref):
    o_ref[:, : _KS] = e_ref[:] + b_ref[:]
    o_ref[:, _KS:] = jnp.zeros((o_ref.shape[0], _KSP - _KS), jnp.float32)


def _bias_table(emb, base_emb):
    rb = 1024
    grid = (_NUM_CTX + rb - 1) // rb
    return pl.pallas_call(
        _bias_body,
        grid=(grid,),
        in_specs=[
            pl.BlockSpec((rb, _KS), lambda i: (i, 0)),
            pl.BlockSpec((1, _KS), lambda i: (0, 0)),
        ],
        out_specs=pl.BlockSpec((rb, _KSP), lambda i: (i, 0)),
        out_shape=jax.ShapeDtypeStruct((_NUM_CTX, _KSP), jnp.float32),
    )(emb, base_emb.reshape(1, _KS))


def _compact_chunk(buf256, buf3):
    def row_body(p, carry):
        for bb in range(_BC):
            r = bb * _NCOLP + p
            for c in range(15):
                buf3[bb, p, pl.ds(c * 16, 16)] = buf256[r, pl.ds(c * 16, 16)]
            buf3[bb, p, pl.ds(_KS - 16, 16)] = buf256[r, pl.ds(_KS - 16, 16)]
        return carry

    lax.fori_loop(0, _NCOL, row_body, 0)


def _sc_gather_body(tab_hbm, idx_hbm, out_hbm, idx_v,
                    buf_a, buf_b, b3_a, b3_b,
                    gsem_a, gsem_b, osem_a, osem_b):
    wid = lax.axis_index("s") * _NC + lax.axis_index("c")
    b0 = wid * _BW
    pltpu.sync_copy(idx_hbm.at[pl.ds(b0 * _NCOLP, _IDXW)], idx_v)

    def idx_at(ch):
        return idx_v.at[pl.ds(ch * _CHI, _CHI)]

    def out_at(ch):
        return out_hbm.at[pl.ds(b0 + ch * _BC, _BC)]

    lanes = ((buf_a, b3_a, gsem_a, osem_a, 0),
             (buf_b, b3_b, gsem_b, osem_b, 1))

    for buf, b3, gsem, osem, off in lanes:
        pltpu.async_copy(tab_hbm.at[idx_at(off)], buf, gsem)

    def body(ch2, carry):
        for buf, b3, gsem, osem, off in lanes:
            ch = ch2 * 2 + off
            pltpu.make_async_copy(tab_hbm.at[idx_at(ch)], buf, gsem).wait()

            @pl.when(ch2 >= 1)
            def _wait_prev_out():
                pltpu.make_async_copy(b3, out_at(ch - 2), osem).wait()

            _compact_chunk(buf, b3)
            pltpu.async_copy(b3, out_at(ch), osem)

            @pl.when(ch2 + 1 < _NCHUNK // 2)
            def _prefetch_next():
                pltpu.async_copy(tab_hbm.at[idx_at(ch + 2)], buf, gsem)
        return carry

    lax.fori_loop(0, _NCHUNK // 2, body, 0)

    for buf, b3, gsem, osem, off in lanes:
        pltpu.make_async_copy(b3, out_at(_NCHUNK - 2 + off), osem).wait()


_sc_gather = functools.partial(
    pl.kernel,
    mesh=plsc.VectorSubcoreMesh(core_axis_name="c", subcore_axis_name="s"),
    out_type=jax.ShapeDtypeStruct((_B, _NCOL, _KS), jnp.float32),
    scratch_types=[
        pltpu.VMEM((_IDXW,), jnp.int32),
        pltpu.VMEM((_CHI, _KSP), jnp.float32),
        pltpu.VMEM((_CHI, _KSP), jnp.float32),
        pltpu.VMEM((_BC, _NCOL, _KS), jnp.float32),
        pltpu.VMEM((_BC, _NCOL, _KS), jnp.float32),
        pltpu.SemaphoreType.DMA,
        pltpu.SemaphoreType.DMA,
        pltpu.SemaphoreType.DMA,
        pltpu.SemaphoreType.DMA,
    ],
)(_sc_gather_body)


def kernel(x, emb, base_emb):
    x = x.astype(jnp.int32)
    inds = _compute_inds(x)
    emb_b = _bias_table(emb, base_emb)
    return _sc_gather(emb_b, inds.reshape(_B * _NCOLP))


# speedup vs baseline: 1.0090x; 1.0090x over previous
import functools

import jax
import jax.numpy as jnp
from jax import lax
from jax.experimental import pallas as pl
from jax.experimental.pallas import tpu as pltpu
from jax.experimental.pallas import tpu_sc as plsc

_K = 6
_KS = 245
_NUM_CTX = sum(5 ** i for i in range(_K + 1))
_B, _L = 4096, 50
_NCOL = _L + 1
_NCOLP = _NCOL + 1
_KSP = 256

_NC, _NS = 2, 16
_NW = _NC * _NS
_BW = _B // _NW
_BC = 2
_NCHUNK = _BW // _BC
_CHI = _BC * _NCOLP
_IDXW = _BW * _NCOLP


def _inds_body(x_ref, out_ref):
    x = x_ref[:]
    offs = [(5 ** m - 1) // 4 for m in range(_K + 1)]
    cols = []
    v = jnp.zeros((_B, 1), jnp.int32)
    cols.append(v + offs[0])
    for i in range(1, _K):
        v = v * 5 + x[:, i - 1:i]
        cols.append(v + offs[i])
    wide = _L - _K + 1
    big = jnp.zeros((_B, wide), jnp.int32)
    for j in range(_K):
        big = big * 5 + x[:, j:j + wide]
    cols.append(big + offs[_K])
    cols.append(jnp.zeros((_B, 1), jnp.int32))
    out_ref[:] = jnp.concatenate(cols, axis=1)


def _compute_inds(x):
    return pl.pallas_call(
        _inds_body,
        out_shape=jax.ShapeDtypeStruct((_B, _NCOLP), jnp.int32),
    )(x)


def _bias_body(e_ref, b_ref, o_ref):
    o_ref[:, : _KS] = e_ref[:] + b_ref[:]
    o_ref[:, _KS:] = jnp.zeros((o_ref.shape[0], _KSP - _KS), jnp.float32)


def _bias_table(emb, base_emb):
    rb = 1024
    grid = (_NUM_CTX + rb - 1) // rb
    return pl.pallas_call(
        _bias_body,
        grid=(grid,),
        in_specs=[
            pl.BlockSpec((rb, _KS), lambda i: (i, 0)),
            pl.BlockSpec((1, _KS), lambda i: (0, 0)),
        ],
        out_specs=pl.BlockSpec((rb, _KSP), lambda i: (i, 0)),
        out_shape=jax.ShapeDtypeStruct((_NUM_CTX, _KSP), jnp.float32),
    )(emb, base_emb.reshape(1, _KS))


def _copy_row(buf256, buf3, bb, p, r):
    for c in range(15):
        buf3[bb, p, pl.ds(c * 16, 16)] = buf256[r, pl.ds(c * 16, 16)]
    buf3[bb, p, pl.ds(_KS - 16, 16)] = buf256[r, pl.ds(_KS - 16, 16)]


def _compact_chunk(buf256, buf3):
    def tile_body(pt, carry):
        for bb in range(_BC):
            for sub in range(8):
                p = pt * 8 + sub
                _copy_row(buf256, buf3, bb, p, bb * _NCOLP + p)
        return carry

    lax.fori_loop(0, _NCOL // 8, tile_body, 0)
    for bb in range(_BC):
        for sub in range(_NCOL - (_NCOL // 8) * 8):
            p = (_NCOL // 8) * 8 + sub
            _copy_row(buf256, buf3, bb, p, bb * _NCOLP + p)


def _sc_gather_body(tab_hbm, idx_hbm, out_hbm, idx_v,
                    buf_a, buf_b, b3_a, b3_b,
                    gsem_a, gsem_b, osem_a, osem_b):
    wid = lax.axis_index("s") * _NC + lax.axis_index("c")
    b0 = wid * _BW
    pltpu.sync_copy(idx_hbm.at[pl.ds(b0 * _NCOLP, _IDXW)], idx_v)

    def idx_at(ch):
        return idx_v.at[pl.ds(ch * _CHI, _CHI)]

    def out_at(ch):
        return out_hbm.at[pl.ds(b0 + ch * _BC, _BC)]

    lanes = ((buf_a, b3_a, gsem_a, osem_a, 0),
             (buf_b, b3_b, gsem_b, osem_b, 1))

    for buf, b3, gsem, osem, off in lanes:
        pltpu.async_copy(tab_hbm.at[idx_at(off)], buf, gsem)

    def body(ch2, carry):
        for buf, b3, gsem, osem, off in lanes:
            ch = ch2 * 2 + off
            pltpu.make_async_copy(tab_hbm.at[idx_at(ch)], buf, gsem).wait()

            @pl.when(ch2 >= 1)
            def _wait_prev_out():
                pltpu.make_async_copy(b3, out_at(ch - 2), osem).wait()

            _compact_chunk(buf, b3)
            pltpu.async_copy(b3, out_at(ch), osem)

            @pl.when(ch2 + 1 < _NCHUNK // 2)
            def _prefetch_next():
                pltpu.async_copy(tab_hbm.at[idx_at(ch + 2)], buf, gsem)
        return carry

    lax.fori_loop(0, _NCHUNK // 2, body, 0)

    for buf, b3, gsem, osem, off in lanes:
        pltpu.make_async_copy(b3, out_at(_NCHUNK - 2 + off), osem).wait()


_sc_gather = functools.partial(
    pl.kernel,
    mesh=plsc.VectorSubcoreMesh(core_axis_name="c", subcore_axis_name="s"),
    out_type=jax.ShapeDtypeStruct((_B, _NCOL, _KS), jnp.float32),
    scratch_types=[
        pltpu.VMEM((_IDXW,), jnp.int32),
        pltpu.VMEM((_CHI, _KSP), jnp.float32),
        pltpu.VMEM((_CHI, _KSP), jnp.float32),
        pltpu.VMEM((_BC, _NCOL, _KS), jnp.float32),
        pltpu.VMEM((_BC, _NCOL, _KS), jnp.float32),
        pltpu.SemaphoreType.DMA,
        pltpu.SemaphoreType.DMA,
        pltpu.SemaphoreType.DMA,
        pltpu.SemaphoreType.DMA,
    ],
)(_sc_gather_body)


def kernel(x, emb, base_emb):
    x = x.astype(jnp.int32)
    inds = _compute_inds(x)
    emb_b = _bias_table(emb, base_emb)
    return _sc_gather(emb_b, inds.reshape(_B * _NCOLP))


# speedup vs baseline: 1.1237x; 1.1138x over previous
import functools

import jax
import jax.numpy as jnp
from jax import lax
from jax.experimental import pallas as pl
from jax.experimental.pallas import tpu as pltpu
from jax.experimental.pallas import tpu_sc as plsc

_K = 6
_KS = 245
_NUM_CTX = sum(5 ** i for i in range(_K + 1))
_B, _L = 4096, 50
_NCOL = _L + 1
_R = _B * _NCOL
_KSP = 256

_NC, _NS = 2, 16
_NW = _NC * _NS
_BW = _B // _NW
_ROWS_W = _BW * _NCOL
_CH = 96
_NCHUNK = _ROWS_W // _CH
_NG = _BW // 16

_P5 = [5 ** i for i in range(_K + 1)]
_OFFS = [(5 ** m - 1) // 4 for m in range(_K + 1)]


def _bias_body(e_ref, b_ref, o_ref):
    o_ref[:, : _KS] = e_ref[:] + b_ref[:]
    o_ref[:, _KS:] = jnp.zeros((o_ref.shape[0], _KSP - _KS), jnp.float32)


def _bias_table(emb, base_emb):
    rb = 1024
    grid = (_NUM_CTX + rb - 1) // rb
    return pl.pallas_call(
        _bias_body,
        grid=(grid,),
        in_specs=[
            pl.BlockSpec((rb, _KS), lambda i: (i, 0)),
            pl.BlockSpec((1, _KS), lambda i: (0, 0)),
        ],
        out_specs=pl.BlockSpec((rb, _KSP), lambda i: (i, 0)),
        out_shape=jax.ShapeDtypeStruct((_NUM_CTX, _KSP), jnp.float32),
    )(emb, base_emb.reshape(1, _KS))


def _compute_indices(x_v, idx_v):
    lanes = lax.iota(jnp.int32, 16)
    for g in range(_NG):
        rows = lanes + g * 16
        cols = [plsc.load_gather(x_v, [rows, jnp.full((16,), j, jnp.int32)])
                for j in range(_L)]
        base = (g * 16) * _NCOL
        tgt0 = lanes * _NCOL + base

        def put(p, vals):
            plsc.store_scatter(idx_v, [tgt0 + p], vals)

        v = jnp.zeros((16,), jnp.int32)
        put(0, v + _OFFS[0])
        for p in range(1, _K):
            v = v * 5 + cols[p - 1]
            put(p, v + _OFFS[p])
        w = v * 5 + cols[_K - 1]
        put(_K, w + _OFFS[_K])
        for p in range(_K + 1, _NCOL):
            w = (w - cols[p - 1 - _K] * _P5[_K - 1]) * 5 + cols[p - 1]
            put(p, w + _OFFS[_K])


def _sc_gather_body(tab_hbm, x_hbm, out_hbm, x_v, idx_v,
                    buf0, buf1, buf2, buf3,
                    gsem0, gsem1, gsem2, gsem3,
                    osem0, osem1, osem2, osem3):
    wid = lax.axis_index("s") * _NC + lax.axis_index("c")
    b0 = wid * _BW
    base = wid * _ROWS_W
    pltpu.sync_copy(x_hbm.at[pl.ds(b0, _BW)], x_v)
    _compute_indices(x_v, idx_v)

    bufs = (buf0, buf1, buf2, buf3)
    gsems = (gsem0, gsem1, gsem2, gsem3)
    osems = (osem0, osem1, osem2, osem3)

    def idx_at(ch):
        return idx_v.at[pl.ds(ch * _CH, _CH)]

    def out_at(ch):
        return out_hbm.at[pl.ds(base + ch * _CH, _CH)]

    pltpu.async_copy(tab_hbm.at[idx_at(0)], buf0, gsem0)
    pltpu.async_copy(tab_hbm.at[idx_at(1)], buf1, gsem1)

    def body(ch4, carry):
        for k in range(4):
            ch = ch4 * 4 + k
            kp = (k + 2) % 4
            pltpu.make_async_copy(tab_hbm.at[idx_at(ch)], bufs[k],
                                  gsems[k]).wait()
            pltpu.async_copy(bufs[k], out_at(ch), osems[k])

            @pl.when(ch + 2 < _NCHUNK)
            def _prefetch():
                @pl.when(ch >= 2)
                def _free_buf():
                    pltpu.make_async_copy(bufs[kp], out_at(ch - 2),
                                          osems[kp]).wait()
                pltpu.async_copy(tab_hbm.at[idx_at(ch + 2)], bufs[kp],
                                 gsems[kp])
        return carry

    lax.fori_loop(0, _NCHUNK // 4, body, 0)

    for k in range(4):
        ch = _NCHUNK - 4 + k
        pltpu.make_async_copy(bufs[ch % 4], out_at(ch), osems[ch % 4]).wait()


_sc_gather = functools.partial(
    pl.kernel,
    mesh=plsc.VectorSubcoreMesh(core_axis_name="c", subcore_axis_name="s"),
    out_type=jax.ShapeDtypeStruct((_R, _KSP), jnp.float32),
    scratch_types=[
        pltpu.VMEM((_BW, _L), jnp.int32),
        pltpu.VMEM((_ROWS_W,), jnp.int32),
        pltpu.VMEM((_CH, _KSP), jnp.float32),
        pltpu.VMEM((_CH, _KSP), jnp.float32),
        pltpu.VMEM((_CH, _KSP), jnp.float32),
        pltpu.VMEM((_CH, _KSP), jnp.float32),
        pltpu.SemaphoreType.DMA,
        pltpu.SemaphoreType.DMA,
        pltpu.SemaphoreType.DMA,
        pltpu.SemaphoreType.DMA,
        pltpu.SemaphoreType.DMA,
        pltpu.SemaphoreType.DMA,
        pltpu.SemaphoreType.DMA,
        pltpu.SemaphoreType.DMA,
    ],
    compiler_params=pltpu.CompilerParams(needs_layout_passes=False),
)(_sc_gather_body)


def kernel(x, emb, base_emb):
    x = x.astype(jnp.int32)
    emb_b = _bias_table(emb, base_emb)
    out = _sc_gather(emb_b, x)
    return out[:, : _KS].reshape(_B, _NCOL, _KS)


# speedup vs baseline: 1.4360x; 1.2779x over previous
import functools

import jax
import jax.numpy as jnp
from jax import lax
from jax.experimental import pallas as pl
from jax.experimental.pallas import tpu as pltpu
from jax.experimental.pallas import tpu_sc as plsc

_K = 6
_KS = 245
_NUM_CTX = sum(5 ** i for i in range(_K + 1))
_B, _L = 4096, 50
_NCOL = _L + 1
_R = _B * _NCOL
_KSP = 256

_NC, _NS = 2, 16
_NW = _NC * _NS
_BW = _B // _NW
_NCOLP = 56
_BC = 1
_NCHUNK = _BW // _BC
_NG = _BW // 16
_IDXW = _BW * _NCOLP

_P5 = [5 ** i for i in range(_K + 1)]
_OFFS = [(5 ** m - 1) // 4 for m in range(_K + 1)]


def _bias_body(e_ref, b_ref, o_ref):
    o_ref[:, : _KS] = e_ref[:] + b_ref[:]
    o_ref[:, _KS:] = jnp.zeros((o_ref.shape[0], _KSP - _KS), jnp.float32)


def _bias_table(emb, base_emb):
    rb = 1024
    grid = (_NUM_CTX + rb - 1) // rb
    return pl.pallas_call(
        _bias_body,
        grid=(grid,),
        in_specs=[
            pl.BlockSpec((rb, _KS), lambda i: (i, 0)),
            pl.BlockSpec((1, _KS), lambda i: (0, 0)),
        ],
        out_specs=pl.BlockSpec((rb, _KSP), lambda i: (i, 0)),
        out_shape=jax.ShapeDtypeStruct((_NUM_CTX, _KSP), jnp.float32),
    )(emb, base_emb.reshape(1, _KS))


def _compute_indices(x_v, idx_v):
    lanes = lax.iota(jnp.int32, 16)
    for g in range(_NG):
        rows = lanes + g * 16
        cols = [plsc.load_gather(x_v, [rows, jnp.full((16,), j, jnp.int32)])
                for j in range(_L)]
        base = (g * 16) * _NCOLP
        tgt0 = lanes * _NCOLP + base

        def put(p, vals):
            plsc.store_scatter(idx_v, [tgt0 + p], vals)

        v = jnp.zeros((16,), jnp.int32)
        put(0, v + _OFFS[0])
        for p in range(1, _K):
            v = v * 5 + cols[p - 1]
            put(p, v + _OFFS[p])
        w = v * 5 + cols[_K - 1]
        put(_K, w + _OFFS[_K])
        for p in range(_K + 1, _NCOL):
            w = (w - cols[p - 1 - _K] * _P5[_K - 1]) * 5 + cols[p - 1]
            put(p, w + _OFFS[_K])


def _sc_gather_body(tab_hbm, x_hbm, out_hbm, x_v, idx_v,
                    buf0, buf1, buf2, buf3,
                    gsem0, gsem1, gsem2, gsem3,
                    osem0, osem1, osem2, osem3):
    wid = lax.axis_index("s") * _NC + lax.axis_index("c")
    b0 = wid * _BW
    pltpu.sync_copy(x_hbm.at[pl.ds(b0, _BW)], x_v)
    _compute_indices(x_v, idx_v)

    bufs = (buf0, buf1, buf2, buf3)
    gsems = (gsem0, gsem1, gsem2, gsem3)
    osems = (osem0, osem1, osem2, osem3)

    def gather_chunk(ch, k):
        for bb in range(_BC):
            lb = ch * _BC + bb
            idx_slice = idx_v.at[pl.ds(lb * _NCOLP, _NCOL)]
            pltpu.async_copy(tab_hbm.at[idx_slice], bufs[k].at[bb], gsems[k])

    def wait_gather(ch, k):
        for bb in range(_BC):
            lb = ch * _BC + bb
            idx_slice = idx_v.at[pl.ds(lb * _NCOLP, _NCOL)]
            pltpu.make_async_copy(tab_hbm.at[idx_slice], bufs[k].at[bb],
                                  gsems[k]).wait()

    def out_at(ch):
        return out_hbm.at[pl.ds(b0 + ch * _BC, _BC)]

    gather_chunk(0, 0)
    gather_chunk(1, 1)

    def body(ch4, carry):
        for k in range(4):
            ch = ch4 * 4 + k
            kp = (k + 2) % 4
            wait_gather(ch, k)
            pltpu.async_copy(bufs[k], out_at(ch), osems[k])

            @pl.when(ch + 2 < _NCHUNK)
            def _prefetch():
                @pl.when(ch >= 2)
                def _free_buf():
                    pltpu.make_async_copy(bufs[kp], out_at(ch - 2),
                                          osems[kp]).wait()
                gather_chunk(ch + 2, kp)
        return carry

    lax.fori_loop(0, _NCHUNK // 4, body, 0)

    for k in range(4):
        ch = _NCHUNK - 4 + k
        pltpu.make_async_copy(bufs[ch % 4], out_at(ch), osems[ch % 4]).wait()


_sc_gather = functools.partial(
    pl.kernel,
    mesh=plsc.VectorSubcoreMesh(core_axis_name="c", subcore_axis_name="s"),
    out_type=jax.ShapeDtypeStruct((_B, _NCOL, _KSP), jnp.float32),
    scratch_types=[
        pltpu.VMEM((_BW, _L), jnp.int32),
        pltpu.VMEM((_IDXW,), jnp.int32),
        pltpu.VMEM((_BC, _NCOL, _KSP), jnp.float32),
        pltpu.VMEM((_BC, _NCOL, _KSP), jnp.float32),
        pltpu.VMEM((_BC, _NCOL, _KSP), jnp.float32),
        pltpu.VMEM((_BC, _NCOL, _KSP), jnp.float32),
        pltpu.SemaphoreType.DMA,
        pltpu.SemaphoreType.DMA,
        pltpu.SemaphoreType.DMA,
        pltpu.SemaphoreType.DMA,
        pltpu.SemaphoreType.DMA,
        pltpu.SemaphoreType.DMA,
        pltpu.SemaphoreType.DMA,
        pltpu.SemaphoreType.DMA,
    ],
    compiler_params=pltpu.CompilerParams(needs_layout_passes=False),
)(_sc_gather_body)


def kernel(x, emb, base_emb):
    x = x.astype(jnp.int32)
    emb_b = _bias_table(emb, base_emb)
    out = _sc_gather(emb_b, x)
    return out[:, :, : _KS]
